# baseline (speedup 1.0000x reference)
import functools

import jax
import jax.numpy as jnp
from jax import lax
from jax.experimental import pallas as pl
from jax.experimental.pallas import tpu as pltpu
from jax.experimental.pallas import tpu_sc as plsc

N_H = 10
L = 16
NC, NS = 2, 16
NW = NC * NS
B, T = 4096, 256
BW = B // NW
GP = 32
NG = GP // L
NPASS = BW // GP

_TC = (0.9993386704758617, -0.3274132062807878, 0.1174902383200023,
       -0.03380254595095054, 0.00660837635036598, -0.0007449281113185158,
       3.58762642613808e-05)
_CLAMP = 2.25


def _tanh_poly(a):
    a = jnp.minimum(jnp.maximum(a, -_CLAMP), _CLAMP)
    u = a * a
    p = jnp.float32(_TC[6])
    for c in _TC[5::-1]:
        p = p * u + jnp.float32(c)
    return a * p


def _rnn_body(xT_hbm, win_hbm, wrec_hbm, out_hbm, x_v, out_v, win_v, wrec_v):
    wid = lax.axis_index("s") * NC + lax.axis_index("c")
    pltpu.sync_copy(win_hbm, win_v)
    pltpu.sync_copy(wrec_hbm, wrec_v)
    pltpu.sync_copy(xT_hbm.at[:, pl.ds(wid * BW, BW)], x_v)

    iota = lax.iota(jnp.int32, L)
    row_idx = [iota + g * L for g in range(NG)]

    def do_pass(p, carry):
        b0 = wid * BW + p * GP

        def step(t, h):
            tj = t * N_H
            xs = [x_v[t, pl.ds(p * GP + g * L, L)] for g in range(NG)]
            new_h = [[None] * N_H for _ in range(NG)]
            for j in range(N_H):
                wj = win_v[j, :]
                accs = [xs[g] * wj for g in range(NG)]
                for i in range(N_H):
                    w = wrec_v[i * N_H + j, :]
                    for g in range(NG):
                        accs[g] = accs[g] + h[g * N_H + i] * w
                col = jnp.broadcast_to(tj + j, (L,))
                for g in range(NG):
                    hv = _tanh_poly(accs[g])
                    new_h[g][j] = hv
                    plsc.store_scatter(out_v, [row_idx[g], col], hv)
            return tuple(new_h[g][j] for g in range(NG) for j in range(N_H))

        h0 = tuple(jnp.zeros((L,), jnp.float32) for _ in range(NG * N_H))
        lax.fori_loop(0, T, step, h0, unroll=False)
        pltpu.sync_copy(out_v, out_hbm.at[pl.ds(b0, GP)])
        return carry

    lax.fori_loop(0, NPASS, do_pass, 0, unroll=False)


@jax.jit
def kernel(x, W_in, W_rec):
    xT = jnp.transpose(x)
    win_b = jnp.broadcast_to(W_in[:, None], (N_H, L))
    wrec_b = jnp.broadcast_to(W_rec.reshape(-1)[:, None], (N_H * N_H, L))

    run = pl.kernel(
        _rnn_body,
        out_type=jax.ShapeDtypeStruct((B, T * N_H), jnp.float32),
        mesh=plsc.VectorSubcoreMesh(core_axis_name="c", subcore_axis_name="s"),
        compiler_params=pltpu.CompilerParams(
            use_tc_tiling_on_sc=False, needs_layout_passes=False),
        scratch_types=[
            pltpu.VMEM((T, BW), jnp.float32),
            pltpu.VMEM((GP, T * N_H), jnp.float32),
            pltpu.VMEM((N_H, L), jnp.float32),
            pltpu.VMEM((N_H * N_H, L), jnp.float32),
        ],
    )
    return run(xT, win_b, wrec_b).reshape(B, T, N_H)


# speedup vs baseline: 1.1730x; 1.1730x over previous
import jax
import jax.numpy as jnp
from jax import lax
from jax.experimental import pallas as pl
from jax.experimental.pallas import tpu as pltpu
from jax.experimental.pallas import tpu_sc as plsc

N_H = 10
L = 16
NC, NS = 2, 16
NW = NC * NS
B, T = 4096, 256
BW = B // NW
GP = 32
NPASS = BW // GP

_TC = (0.9993386704758617, -0.3274132062807878, 0.1174902383200023,
       -0.03380254595095054, 0.00660837635036598, -0.0007449281113185158,
       3.58762642613808e-05)
_CLAMP = 2.25


def _tanh_poly(a):
    dt = a.dtype
    a = jnp.minimum(jnp.maximum(a, jnp.asarray(-_CLAMP, dt)),
                    jnp.asarray(_CLAMP, dt))
    c0, c1, c2, c3, c4, c5, c6 = (jnp.asarray(c, dt) for c in _TC)
    u = a * a
    u2 = u * u
    u4 = u2 * u2
    p01 = c0 + c1 * u
    p23 = c2 + c3 * u
    p45 = c4 + c5 * u
    return a * (p01 + u2 * p23 + u4 * (p45 + u2 * c6))


def _tree_sum(prods):
    while len(prods) > 1:
        nxt = [prods[k] + prods[k + 1] for k in range(0, len(prods) - 1, 2)]
        if len(prods) % 2:
            nxt.append(prods[-1])
        prods = nxt
    return prods[0]


def _rnn_body(xT_hbm, win_hbm, wrec_hbm, out_hbm, x_v, out_v, win_v, wrec_v):
    wid = lax.axis_index("s") * NC + lax.axis_index("c")
    pltpu.sync_copy(win_hbm, win_v)
    pltpu.sync_copy(wrec_hbm, wrec_v)
    pltpu.sync_copy(xT_hbm.at[:, pl.ds(wid * BW, BW)], x_v)

    iota = lax.iota(jnp.int32, L)
    row_even = iota * 2
    row_odd = iota * 2 + 1

    def do_pass(p, carry):
        b0 = wid * BW + p * GP

        def step(t, h):
            tj = t * N_H
            xv = x_v[t, pl.ds(p * GP, GP)]
            new_h = [None] * N_H
            for j in range(N_H):
                wj = win_v[j, :]
                prods = [xv * wj] + [h[i] * wrec_v[i * N_H + j, :]
                                     for i in range(N_H)]
                hv = _tanh_poly(_tree_sum(prods))
                new_h[j] = hv
                ha, hb = plsc.unpack(hv, format=plsc.PackFormat.INTERLEAVED)
                col = jnp.broadcast_to(tj + j, (L,))
                plsc.store_scatter(out_v, [row_even, col], ha)
                plsc.store_scatter(out_v, [row_odd, col], hb)
            return tuple(new_h)

        h0 = tuple(jnp.zeros((2 * L,), jnp.bfloat16) for _ in range(N_H))
        lax.fori_loop(0, T, step, h0, unroll=False)
        pltpu.sync_copy(out_v, out_hbm.at[pl.ds(b0, GP)])
        return carry

    lax.fori_loop(0, NPASS, do_pass, 0, unroll=False)


@jax.jit
def kernel(x, W_in, W_rec):
    xT = jnp.transpose(x).astype(jnp.bfloat16)
    win_b = jnp.broadcast_to(W_in.astype(jnp.bfloat16)[:, None], (N_H, 2 * L))
    wrec_b = jnp.broadcast_to(
        W_rec.reshape(-1).astype(jnp.bfloat16)[:, None], (N_H * N_H, 2 * L))

    run = pl.kernel(
        _rnn_body,
        out_type=jax.ShapeDtypeStruct((B, T * N_H), jnp.float32),
        mesh=plsc.VectorSubcoreMesh(core_axis_name="c", subcore_axis_name="s"),
        compiler_params=pltpu.CompilerParams(
            use_tc_tiling_on_sc=False, needs_layout_passes=False),
        scratch_types=[
            pltpu.VMEM((T, BW), jnp.bfloat16),
            pltpu.VMEM((GP, T * N_H), jnp.float32),
            pltpu.VMEM((N_H, 2 * L), jnp.bfloat16),
            pltpu.VMEM((N_H * N_H, 2 * L), jnp.bfloat16),
        ],
    )
    return run(xT, win_b, wrec_b).reshape(B, T, N_H)


# speedup vs baseline: 1.4872x; 1.2679x over previous
import jax
import jax.numpy as jnp
from jax import lax
from jax.experimental import pallas as pl
from jax.experimental.pallas import tpu as pltpu
from jax.experimental.pallas import tpu_sc as plsc

N_H = 10
L = 16
NC, NS = 2, 16
NW = NC * NS
B, T = 4096, 256
BW = B // NW
GP = 32
NPASS = BW // GP

_TC = (0.9993386704758617, -0.3274132062807878, 0.1174902383200023,
       -0.03380254595095054, 0.00660837635036598, -0.0007449281113185158,
       3.58762642613808e-05)
_CLAMP = 2.25


def _tanh_poly(a):
    dt = a.dtype
    a = jnp.minimum(jnp.maximum(a, jnp.asarray(-_CLAMP, dt)),
                    jnp.asarray(_CLAMP, dt))
    c0, c1, c2, c3, c4, c5, c6 = (jnp.asarray(c, dt) for c in _TC)
    u = a * a
    u2 = u * u
    u4 = u2 * u2
    p01 = c0 + c1 * u
    p23 = c2 + c3 * u
    p45 = c4 + c5 * u
    return a * (p01 + u2 * p23 + u4 * (p45 + u2 * c6))


def _tree_sum(prods):
    while len(prods) > 1:
        nxt = [prods[k] + prods[k + 1] for k in range(0, len(prods) - 1, 2)]
        if len(prods) % 2:
            nxt.append(prods[-1])
        prods = nxt
    return prods[0]


_GDN = lax.GatherDimensionNumbers(
    offset_dims=(), collapsed_slice_dims=(0,), start_index_map=(0,))
NWREG = (N_H * N_H + N_H + L - 1) // L


def _rnn_body(xT_hbm, wpack_hbm, out_hbm, x_v, out_v, wpack_v):
    wid = lax.axis_index("s") * NC + lax.axis_index("c")
    pltpu.sync_copy(wpack_hbm, wpack_v)
    pltpu.sync_copy(xT_hbm.at[:, pl.ds(wid * BW, BW)], x_v)

    iota = lax.iota(jnp.int32, L)
    row_even = iota * 2
    row_odd = iota * 2 + 1

    wregs = [wpack_v[r, :] for r in range(NWREG)]

    def wsplat(k):
        r, lane = divmod(k, L)
        idx = jnp.full((L, 1), lane, jnp.int32)
        w32 = lax.gather(wregs[r], idx, _GDN, (1,),
                         mode=lax.GatherScatterMode.PROMISE_IN_BOUNDS)
        return plsc.bitcast(w32, jnp.bfloat16)

    def do_pass(p, carry):
        b0 = wid * BW + p * GP

        def step(t, h):
            tj = t * N_H
            xv = x_v[t, pl.ds(p * GP, GP)]
            new_h = [None] * N_H
            for j in range(N_H):
                prods = [xv * wsplat(j)] + [h[i] * wsplat(N_H + i * N_H + j)
                                            for i in range(N_H)]
                hv = _tanh_poly(_tree_sum(prods))
                new_h[j] = hv
                ha, hb = plsc.unpack(hv, format=plsc.PackFormat.INTERLEAVED)
                col = jnp.broadcast_to(tj + j, (L,))
                plsc.store_scatter(out_v, [row_even, col], ha)
                plsc.store_scatter(out_v, [row_odd, col], hb)
            return tuple(new_h)

        h0 = tuple(jnp.zeros((2 * L,), jnp.bfloat16) for _ in range(N_H))
        lax.fori_loop(0, T, step, h0, unroll=False)
        pltpu.sync_copy(out_v, out_hbm.at[pl.ds(b0, GP)])
        return carry

    lax.fori_loop(0, NPASS, do_pass, 0, unroll=False)


@jax.jit
def kernel(x, W_in, W_rec):
    xT = jnp.transpose(x).astype(jnp.bfloat16)
    w_all = jnp.concatenate([W_in, W_rec.reshape(-1)]).astype(jnp.bfloat16)
    w_u32 = lax.bitcast_convert_type(w_all, jnp.uint16).astype(jnp.uint32)
    w_dup = (w_u32 << 16) | w_u32
    wpack = jnp.pad(w_dup, (0, NWREG * L - w_dup.shape[0])).reshape(NWREG, L)

    run = pl.kernel(
        _rnn_body,
        out_type=jax.ShapeDtypeStruct((B, T * N_H), jnp.float32),
        mesh=plsc.VectorSubcoreMesh(core_axis_name="c", subcore_axis_name="s"),
        compiler_params=pltpu.CompilerParams(
            use_tc_tiling_on_sc=False, needs_layout_passes=False),
        scratch_types=[
            pltpu.VMEM((T, BW), jnp.bfloat16),
            pltpu.VMEM((GP, T * N_H), jnp.float32),
            pltpu.VMEM((NWREG, L), jnp.uint32),
        ],
    )
    return run(xT, wpack).reshape(B, T, N_H)
